# baseline (speedup 1.0000x reference)
import functools

import jax
import jax.numpy as jnp
from jax import lax
from jax.experimental import pallas as pl
from jax.experimental.pallas import tpu as pltpu
from jax.experimental.pallas import tpu_sc as plsc

N_NODES = 10000
D = 128
BN_EPS = 1e-5

NW = 32
N_PAD = 10240
ROWS_PER_TILE = N_PAD // 16
CHUNK = 128
NBUF = 2


def _make_sc_agg(n_chunks_total):
    n_chunks = n_chunks_total // NW
    mesh = plsc.VectorSubcoreMesh(core_axis_name="c", subcore_axis_name="s")

    @functools.partial(
        pl.kernel,
        out_type=jax.ShapeDtypeStruct((2, N_PAD, D), jnp.float32),
        mesh=mesh,
        scratch_types=[
            pltpu.VMEM((2, CHUNK), jnp.int32),
            pltpu.VMEM((2, CHUNK), jnp.int32),
            pltpu.VMEM((CHUNK, D), jnp.float32),
            pltpu.VMEM((CHUNK, D), jnp.float32),
            pltpu.VMEM_SHARED((N_PAD, D), jnp.float32),
            pltpu.SemaphoreType.DMA,
            pltpu.SemaphoreType.DMA,
            pltpu.SemaphoreType.DMA,
            pltpu.SemaphoreType.DMA,
        ],
    )
    def sc_agg(feats_hbm, idx_hbm, out_hbm,
               idx_v0, idx_v1, rows_v0, rows_v1, acc_sh,
               sem_i0, sem_i1, sem_g0, sem_g1):
        cid = lax.axis_index("c")
        sid = lax.axis_index("s")
        wid = sid * 2 + cid

        idx_bufs = (idx_v0, idx_v1)
        row_bufs = (rows_v0, rows_v1)
        sem_i = (sem_i0, sem_i1)
        sem_g = (sem_g0, sem_g1)

        def start_idx(k, b):
            pltpu.make_async_copy(idx_hbm.at[k], idx_bufs[b], sem_i[b]).start()

        def wait_idx(b):
            pltpu.make_async_copy(idx_hbm.at[0], idx_bufs[b], sem_i[b]).wait()

        def start_gather(b):
            pltpu.make_async_copy(
                feats_hbm.at[idx_bufs[b].at[0]], row_bufs[b], sem_g[b]
            ).start()

        def wait_gather(b):
            pltpu.make_async_copy(
                feats_hbm.at[idx_bufs[b].at[0]], row_bufs[b], sem_g[b]
            ).wait()

        def zrow(i, _):
            for j in range(D // 16):
                rows_v0[i, pl.ds(j * 16, 16)] = jnp.zeros((16,), jnp.float32)
            return 0

        lax.fori_loop(0, CHUNK, zrow, 0)
        for j in range(ROWS_PER_TILE // CHUNK):
            pltpu.sync_copy(
                rows_v0, acc_sh.at[pl.ds(sid * ROWS_PER_TILE + j * CHUNK, CHUNK)]
            )
        plsc.subcore_barrier()

        kb = wid * n_chunks

        start_idx(kb, 0)
        start_idx(kb + 1, 1)
        wait_idx(0)
        start_gather(0)

        def body(g, _):
            for b in range(NBUF):
                c = NBUF * g + b
                o = 1 - b
                wait_gather(b)

                @pl.when(c + 1 < n_chunks)
                def _():
                    wait_idx(o)
                    start_gather(o)

                pltpu.sync_copy(row_bufs[b], acc_sh.at[idx_bufs[b].at[1]], add=True)

                @pl.when(c + 2 < n_chunks)
                def _():
                    start_idx(kb + c + 2, b)

            return 0

        lax.fori_loop(0, n_chunks // NBUF, body, 0)
        plsc.subcore_barrier()

        pltpu.sync_copy(
            acc_sh.at[pl.ds(sid * ROWS_PER_TILE, ROWS_PER_TILE)],
            out_hbm.at[cid, pl.ds(sid * ROWS_PER_TILE, ROWS_PER_TILE)],
        )

    return sc_agg


def _tc_dense_body(agg2_ref, feats_ref, w_ref, b_ref, wr_ref, br_ref, g_ref, bt_ref, out_ref):
    agg = (agg2_ref[0] + agg2_ref[1])[:N_NODES]
    h = jnp.maximum(
        jax.lax.dot(agg, w_ref[...], preferred_element_type=jnp.float32) + b_ref[...],
        0.0,
    )
    res = jnp.maximum(
        jax.lax.dot(feats_ref[...], wr_ref[...], preferred_element_type=jnp.float32)
        + br_ref[...],
        0.0,
    )
    h = h + res
    mean = jnp.mean(h, axis=0, keepdims=True)
    c = h - mean
    var = jnp.mean(c * c, axis=0, keepdims=True)
    out_ref[...] = c * jax.lax.rsqrt(var + BN_EPS) * g_ref[...] + bt_ref[...]


def kernel(feats, edge_index, W, b, W_res, b_res, gamma, beta):
    e = edge_index.shape[1]
    ei = edge_index.astype(jnp.int32)
    e_pad = -(-e // (NW * CHUNK * NBUF)) * (NW * CHUNK * NBUF)
    pad = e_pad - e
    src = jnp.concatenate([ei[0], jnp.zeros((pad,), jnp.int32)])
    dst = jnp.concatenate([ei[1], jnp.full((pad,), N_NODES, jnp.int32)])
    idx_il = jnp.stack([src.reshape(-1, CHUNK), dst.reshape(-1, CHUNK)], axis=1)

    agg2 = _make_sc_agg(idx_il.shape[0])(feats, idx_il)

    return pl.pallas_call(
        _tc_dense_body,
        out_shape=jax.ShapeDtypeStruct((N_NODES, D), jnp.float32),
    )(
        agg2,
        feats,
        W,
        b.reshape(1, D),
        W_res,
        b_res.reshape(1, D),
        gamma.reshape(1, D),
        beta.reshape(1, D),
    )
